# initial kernel scaffold (unmeasured)
import jax
import jax.numpy as jnp
from jax import lax
from jax.experimental import pallas as pl
from jax.experimental.pallas import tpu as pltpu

N_DEV = 16
E_LOC = 4
CAP = 204
EPAD = 128


def kernel(x, router_W, route_idx, expert_W):
    T, D = x.shape
    E, _, H = expert_W.shape

    def body(x_ref, idx_ref, w_ref, out_ref, comm_w, comm_h,
             w_send, w_recv, h_send, h_recv, credit):
        me = lax.axis_index("i")
        left = lax.rem(me + N_DEV - 1, N_DEV)
        right = lax.rem(me + 1, N_DEV)

        xb = x_ref[:, :].astype(jnp.bfloat16)
        e = idx_ref[:, :]

        cols = lax.broadcasted_iota(jnp.int32, (T, EPAD), 1)
        onehot = (e == cols).astype(jnp.int32)
        hist = jnp.sum(onehot, axis=0, keepdims=True)
        incl = jnp.cumsum(onehot, axis=0)
        rank = jnp.sum(onehot * incl, axis=1, keepdims=True) - 1

        comm_w[0, :, :, :] = w_ref[:, :, :].astype(jnp.bfloat16)
        comm_h[0, :, :] = jnp.broadcast_to(hist, (8, EPAD))

        def contrib(origin, slot):
            acc = jnp.zeros((T, H), jnp.float32)
            for j in range(E_LOC):
                xm = jnp.where(e == origin * E_LOC + j, xb,
                               jnp.zeros_like(xb))
                acc = acc + jnp.dot(xm, comm_w[slot, j, :, :],
                                    preferred_element_type=jnp.float32)
            return acc

        y = jnp.zeros((T, H), jnp.float32)
        offs = jnp.zeros((1, EPAD), jnp.int32)

        for h in range(N_DEV - 1):
            s_slot, r_slot = h % 2, (h + 1) % 2
            rw = pltpu.make_async_remote_copy(
                src_ref=comm_w.at[s_slot], dst_ref=comm_w.at[r_slot],
                send_sem=w_send.at[h], recv_sem=w_recv.at[h],
                device_id=(right,), device_id_type=pl.DeviceIdType.MESH)
            rh = pltpu.make_async_remote_copy(
                src_ref=comm_h.at[s_slot], dst_ref=comm_h.at[r_slot],
                send_sem=h_send.at[h], recv_sem=h_recv.at[h],
                device_id=(right,), device_id_type=pl.DeviceIdType.MESH)
            if h >= 1:
                pl.semaphore_wait(credit, 1)
            rw.start()
            rh.start()
            if h == 0:
                y = y + contrib(me, 0)
            rw.wait_send()
            rh.wait_send()
            if h <= N_DEV - 3:
                pl.semaphore_signal(credit, inc=1, device_id=(left,),
                                    device_id_type=pl.DeviceIdType.MESH)
            rw.wait_recv()
            rh.wait_recv()
            origin = lax.rem(me + N_DEV - h - 1, N_DEV)
            y = y + contrib(origin, r_slot)
            offs = offs + jnp.where(origin < me,
                                    comm_h[r_slot, 0:1, :],
                                    jnp.zeros((1, EPAD), jnp.int32))

        gofs = jnp.sum(onehot * offs, axis=1, keepdims=True)
        keep = (gofs + rank) < CAP
        out_ref[:, :] = jnp.where(keep, y, jnp.zeros((T, H), jnp.float32))

    return pl.pallas_call(
        body,
        out_shape=jax.ShapeDtypeStruct((T, H), jnp.float32),
        in_specs=[pl.BlockSpec(memory_space=pltpu.VMEM)] * 3,
        out_specs=pl.BlockSpec(memory_space=pltpu.VMEM),
        scratch_shapes=[
            pltpu.VMEM((2, E, D, H), jnp.bfloat16),
            pltpu.VMEM((2, 8, EPAD), jnp.int32),
            pltpu.SemaphoreType.DMA((N_DEV - 1,)),
            pltpu.SemaphoreType.DMA((N_DEV - 1,)),
            pltpu.SemaphoreType.DMA((N_DEV - 1,)),
            pltpu.SemaphoreType.DMA((N_DEV - 1,)),
            pltpu.SemaphoreType.REGULAR,
        ],
    )(x, route_idx, expert_W)


# baseline (device time: 858437 ns/iter reference)
import jax
import jax.numpy as jnp
from jax import lax
from jax.experimental import pallas as pl
from jax.experimental.pallas import tpu as pltpu

N_DEV = 16
E_LOC = 4
CAP = 204
EPAD = 128


def kernel(x, router_W, route_idx, expert_W):
    T, D = x.shape
    E, _, H = expert_W.shape

    def body(x_ref, idx_ref, w_ref, out_ref, comm_w, comm_h,
             w_send, w_recv, h_send, h_recv, credit):
        me = lax.axis_index("i")
        left = lax.rem(me + N_DEV - 1, N_DEV)
        right = lax.rem(me + 1, N_DEV)

        xb = x_ref[:, :].astype(jnp.bfloat16)
        e = idx_ref[:, :]

        cols = lax.broadcasted_iota(jnp.int32, (T, EPAD), 1)
        onehot = (e == cols).astype(jnp.int32)
        hist = jnp.sum(onehot, axis=0, keepdims=True)
        row = lax.broadcasted_iota(jnp.int32, (T, T), 0)
        col = lax.broadcasted_iota(jnp.int32, (T, T), 1)
        tril = (row >= col).astype(jnp.float32)
        incl = jnp.dot(tril, onehot.astype(jnp.float32),
                       preferred_element_type=jnp.float32).astype(jnp.int32)
        rank = jnp.sum(onehot * incl, axis=1, keepdims=True) - 1

        comm_w[0, :, :, :] = w_ref[:, :, :].astype(jnp.bfloat16)
        comm_h[0, :, :] = jnp.broadcast_to(hist, (8, EPAD))

        def add_contrib(origin, slot):
            for j in range(E_LOC):
                xm = jnp.where(e == origin * E_LOC + j, xb,
                               jnp.zeros_like(xb))
                out_ref[:, :] = out_ref[:, :] + jnp.dot(
                    xm, comm_w[slot, j, :, :],
                    preferred_element_type=jnp.float32)

        out_ref[:, :] = jnp.zeros((T, H), jnp.float32)
        offs = jnp.zeros((1, EPAD), jnp.int32)

        for h in range(N_DEV - 1):
            s_slot, r_slot = h % 2, (h + 1) % 2
            rw = pltpu.make_async_remote_copy(
                src_ref=comm_w.at[s_slot], dst_ref=comm_w.at[r_slot],
                send_sem=w_send.at[h], recv_sem=w_recv.at[h],
                device_id=(right,), device_id_type=pl.DeviceIdType.MESH)
            rh = pltpu.make_async_remote_copy(
                src_ref=comm_h.at[s_slot], dst_ref=comm_h.at[r_slot],
                send_sem=h_send.at[h], recv_sem=h_recv.at[h],
                device_id=(right,), device_id_type=pl.DeviceIdType.MESH)
            if h >= 1:
                pl.semaphore_wait(credit, 1)
            rw.start()
            rh.start()
            if h == 0:
                add_contrib(me, 0)
            rw.wait_send()
            rh.wait_send()
            if h <= N_DEV - 3:
                pl.semaphore_signal(credit, inc=1, device_id=(left,),
                                    device_id_type=pl.DeviceIdType.MESH)
            rw.wait_recv()
            rh.wait_recv()
            origin = lax.rem(me + N_DEV - h - 1, N_DEV)
            add_contrib(origin, r_slot)
            offs = offs + jnp.where(origin < me,
                                    comm_h[r_slot, 0:1, :],
                                    jnp.zeros((1, EPAD), jnp.int32))

        gofs = jnp.sum(onehot * offs, axis=1, keepdims=True)
        keep = (gofs + rank) < CAP
        out_ref[:, :] = jnp.where(keep, out_ref[:, :],
                                  jnp.zeros((T, H), jnp.float32))

    return pl.pallas_call(
        body,
        out_shape=jax.ShapeDtypeStruct((T, H), jnp.float32),
        in_specs=[pl.BlockSpec(memory_space=pltpu.VMEM)] * 3,
        out_specs=pl.BlockSpec(memory_space=pltpu.VMEM),
        scratch_shapes=[
            pltpu.VMEM((2, E, D, H), jnp.bfloat16),
            pltpu.VMEM((2, 8, EPAD), jnp.int32),
            pltpu.SemaphoreType.DMA((N_DEV - 1,)),
            pltpu.SemaphoreType.DMA((N_DEV - 1,)),
            pltpu.SemaphoreType.DMA((N_DEV - 1,)),
            pltpu.SemaphoreType.DMA((N_DEV - 1,)),
            pltpu.SemaphoreType.REGULAR,
        ],
    )(x, route_idx, expert_W)


# device time: 392107 ns/iter; 2.1893x vs baseline; 2.1893x over previous
import jax
import jax.numpy as jnp
from jax import lax
from jax.experimental import pallas as pl
from jax.experimental.pallas import tpu as pltpu

N_DEV = 16
E_LOC = 4
CAP = 204
EPAD = 128
HOPS = N_DEV // 2


def kernel(x, router_W, route_idx, expert_W):
    T, D = x.shape
    E, _, H = expert_W.shape

    def body(x_ref, idx_ref, w_ref, out_ref,
             fw_buf, bw_buf, my_hist, hist_all,
             fw_send, fw_recv, bw_send, bw_recv,
             hs_send, hs_recv, f_credit, b_credit):
        me = lax.axis_index("i")
        left = lax.rem(me + N_DEV - 1, N_DEV)
        right = lax.rem(me + 1, N_DEV)

        xb = x_ref[:, :].astype(jnp.bfloat16)
        e = idx_ref[:, :]

        cols = lax.broadcasted_iota(jnp.int32, (T, EPAD), 1)
        onehot = (e == cols).astype(jnp.int32)
        hist = jnp.sum(onehot, axis=0, keepdims=True)
        my_hist[:, :] = jnp.broadcast_to(hist, (8, EPAD))

        hsend = []
        for d in range(1, N_DEV):
            r = pltpu.make_async_remote_copy(
                src_ref=my_hist, dst_ref=hist_all.at[d],
                send_sem=hs_send.at[d], recv_sem=hs_recv.at[d],
                device_id=(lax.rem(me + d, N_DEV),),
                device_id_type=pl.DeviceIdType.MESH)
            r.start()
            hsend.append(r)

        row = lax.broadcasted_iota(jnp.int32, (T, T), 0)
        col = lax.broadcasted_iota(jnp.int32, (T, T), 1)
        tril = (row >= col).astype(jnp.float32)
        incl = jnp.dot(tril, onehot.astype(jnp.float32),
                       preferred_element_type=jnp.float32).astype(jnp.int32)
        rank = jnp.sum(onehot * incl, axis=1, keepdims=True) - 1

        wb16 = w_ref[:, :, :].astype(jnp.bfloat16)
        fw_buf[0, :, :, :] = wb16
        bw_buf[0, :, :, :] = wb16

        def mk(buf, ssem, rsem, h, dst, half=None):
            if half is None:
                src, dd = buf.at[h % 2], buf.at[(h + 1) % 2]
            else:
                src = buf.at[h % 2, pl.ds(half * 2, 2)]
                dd = buf.at[(h + 1) % 2, pl.ds(half * 2, 2)]
            return pltpu.make_async_remote_copy(
                src_ref=src, dst_ref=dd,
                send_sem=ssem.at[h], recv_sem=rsem.at[h],
                device_id=(dst,), device_id_type=pl.DeviceIdType.MESH)

        fw = [mk(fw_buf, fw_send, fw_recv, h, right,
                 half=(0 if h == HOPS - 1 else None)) for h in range(HOPS)]
        bw = [mk(bw_buf, bw_send, bw_recv, h, left,
                 half=(1 if h == HOPS - 1 else None)) for h in range(HOPS)]

        def add_contrib(origin, buf, slot, experts):
            for j in experts:
                xm = jnp.where(e == origin * E_LOC + j, xb,
                               jnp.zeros_like(xb))
                out_ref[:, :] = out_ref[:, :] + jnp.dot(
                    xm, buf[slot, j, :, :],
                    preferred_element_type=jnp.float32)

        out_ref[:, :] = jnp.zeros((T, H), jnp.float32)

        fw[0].start()
        bw[0].start()
        add_contrib(me, fw_buf, 0, range(E_LOC))

        for h in range(HOPS):
            r_slot = (h + 1) % 2
            fw[h].wait_send()
            if h <= HOPS - 2:
                pl.semaphore_signal(f_credit, inc=1, device_id=(left,),
                                    device_id_type=pl.DeviceIdType.MESH)
            fw[h].wait_recv()
            bw[h].wait_send()
            if h <= HOPS - 2:
                pl.semaphore_signal(b_credit, inc=1, device_id=(right,),
                                    device_id_type=pl.DeviceIdType.MESH)
            bw[h].wait_recv()
            if h + 1 < HOPS:
                pl.semaphore_wait(f_credit, 1)
                fw[h + 1].start()
                pl.semaphore_wait(b_credit, 1)
                bw[h + 1].start()
            of = lax.rem(me + N_DEV - h - 1, N_DEV)
            ob = lax.rem(me + h + 1, N_DEV)
            if h < HOPS - 1:
                add_contrib(of, fw_buf, r_slot, range(E_LOC))
                add_contrib(ob, bw_buf, r_slot, range(E_LOC))
            else:
                add_contrib(of, fw_buf, r_slot, range(0, 2))
                add_contrib(ob, bw_buf, r_slot, range(2, 4))

        offs = jnp.zeros((1, EPAD), jnp.int32)
        for d in range(1, N_DEV):
            hsend[d - 1].wait_recv()
            origin = lax.rem(me + N_DEV - d, N_DEV)
            offs = offs + jnp.where(origin < me, hist_all[d, 0:1, :],
                                    jnp.zeros((1, EPAD), jnp.int32))
        for r in hsend:
            r.wait_send()

        gofs = jnp.sum(onehot * offs, axis=1, keepdims=True)
        keep = (gofs + rank) < CAP
        out_ref[:, :] = jnp.where(keep, out_ref[:, :],
                                  jnp.zeros((T, H), jnp.float32))

    return pl.pallas_call(
        body,
        out_shape=jax.ShapeDtypeStruct((T, H), jnp.float32),
        in_specs=[pl.BlockSpec(memory_space=pltpu.VMEM)] * 3,
        out_specs=pl.BlockSpec(memory_space=pltpu.VMEM),
        scratch_shapes=[
            pltpu.VMEM((2, E, D, H), jnp.bfloat16),
            pltpu.VMEM((2, E, D, H), jnp.bfloat16),
            pltpu.VMEM((8, EPAD), jnp.int32),
            pltpu.VMEM((N_DEV, 8, EPAD), jnp.int32),
            pltpu.SemaphoreType.DMA((HOPS,)),
            pltpu.SemaphoreType.DMA((HOPS,)),
            pltpu.SemaphoreType.DMA((HOPS,)),
            pltpu.SemaphoreType.DMA((HOPS,)),
            pltpu.SemaphoreType.DMA((N_DEV,)),
            pltpu.SemaphoreType.DMA((N_DEV,)),
            pltpu.SemaphoreType.REGULAR,
            pltpu.SemaphoreType.REGULAR,
        ],
    )(x, route_idx, expert_W)
